# baseline (speedup 1.0000x reference)
import functools

import jax
import jax.numpy as jnp
from jax import lax
from jax.experimental import pallas as pl
from jax.experimental.pallas import tpu as pltpu
from jax.experimental.pallas import tpu_sc as plsc

N = 10000
NP = 10240
DIN = 256
H = 512
HC = 128
NCH = H // HC
E = 160000
NC = 2
NS = 16
NW = NC * NS
EPT = 5120
EP = NW * EPT
EB = 128
NB = EPT // EB
RPT = NP // NS

EBG = 32
NBG = EPT // EBG

BR = 1280
GRID_N = NP // BR
BE = 2048
GRID_E = EP // BE

_mesh = plsc.VectorSubcoreMesh(core_axis_name="c", subcore_axis_name="s",
                               num_cores=NC, num_subcores=NS)


def _hist_body(dst_hbm, ones_hbm, zv_hbm, hist_hbm, idx_v, ones_v, acc):
    cid = lax.axis_index("c")
    sid = lax.axis_index("s")
    wid = sid * NC + cid
    pltpu.sync_copy(dst_hbm.at[wid], idx_v)
    pltpu.sync_copy(ones_hbm, ones_v)
    pltpu.sync_copy(zv_hbm.at[pl.ds(sid * RPT, RPT)],
                    acc.at[pl.ds(sid * RPT, RPT)])
    plsc.subcore_barrier()

    @pl.loop(0, NB)
    def _batches(j):
        pltpu.sync_copy(ones_v, acc.at[idx_v.at[j]], add=True)

    plsc.subcore_barrier()
    pltpu.sync_copy(acc.at[pl.ds(sid * RPT, RPT)],
                    hist_hbm.at[cid, pl.ds(sid * RPT, RPT)])


_hist = pl.kernel(
    _hist_body,
    out_type=jax.ShapeDtypeStruct((NC, NP, HC), jnp.float32),
    mesh=_mesh,
    scratch_types=[
        pltpu.VMEM((NB, EB), jnp.int32),
        pltpu.VMEM((EB, HC), jnp.float32),
        pltpu.VMEM_SHARED((NP, HC), jnp.float32),
    ],
)


def _scatter_body(g0, g1, g2, g3, src_hbm, dst_hbm, z_hbm, s_hbm,
                  idxs_v, idxd_v, bufa, bufb, acc, sema, semb):
    cid = lax.axis_index("c")
    sid = lax.axis_index("s")

    @pl.when(cid == 0)
    def _():
        def run_row(gc, wid):
            pltpu.sync_copy(src_hbm.at[wid], idxs_v)
            pltpu.sync_copy(dst_hbm.at[wid], idxd_v)
            pltpu.async_copy(gc.at[idxs_v.at[0]], bufa, sema)

            @pl.loop(0, NB // 2)
            def _batches(j2):
                j = j2 * 2
                pltpu.make_async_copy(gc.at[idxs_v.at[j]], bufa, sema).wait()
                pltpu.async_copy(gc.at[idxs_v.at[j + 1]], bufb, semb)
                pltpu.sync_copy(bufa, acc.at[idxd_v.at[j]], add=True)
                pltpu.make_async_copy(gc.at[idxs_v.at[j + 1]], bufb,
                                      semb).wait()

                @pl.when(j2 < NB // 2 - 1)
                def _():
                    pltpu.async_copy(gc.at[idxs_v.at[j + 2]], bufa, sema)

                pltpu.sync_copy(bufb, acc.at[idxd_v.at[j + 1]], add=True)

        for c, gc in enumerate((g0, g1, g2, g3)):
            pltpu.sync_copy(z_hbm.at[pl.ds(sid * RPT, RPT)],
                            acc.at[pl.ds(sid * RPT, RPT)])
            plsc.subcore_barrier()
            run_row(gc, sid * 2)
            run_row(gc, sid * 2 + 1)
            plsc.subcore_barrier()
            pltpu.sync_copy(acc.at[pl.ds(sid * RPT, RPT)],
                            s_hbm.at[c, pl.ds(sid * RPT, RPT)])
            plsc.subcore_barrier()


_scatter = pl.kernel(
    _scatter_body,
    out_type=jax.ShapeDtypeStruct((NCH, NP, HC), jnp.float32),
    mesh=_mesh,
    scratch_types=[
        pltpu.VMEM((NB, EB), jnp.int32),
        pltpu.VMEM((NB, EB), jnp.int32),
        pltpu.VMEM((EB, HC), jnp.float32),
        pltpu.VMEM((EB, HC), jnp.float32),
        pltpu.VMEM_SHARED((NP, HC), jnp.float32),
        pltpu.SemaphoreType.DMA,
        pltpu.SemaphoreType.DMA,
    ],
)


def _gather_body(p_hbm, q_hbm, src_hbm, dst_hbm, rp_hbm, rq_hbm,
                 idxs_v, idxd_v, bufpa, bufqa, bufpb, bufqb,
                 gpa, gqa, gpb, gqb, wpa, wqa, wpb, wqb):
    cid = lax.axis_index("c")
    sid = lax.axis_index("s")
    wid = sid * NC + cid
    pltpu.sync_copy(src_hbm.at[wid], idxs_v)
    pltpu.sync_copy(dst_hbm.at[wid], idxd_v)
    base = wid * EPT

    def _start_g(j, bp, bq, sp, sq):
        pltpu.async_copy(p_hbm.at[idxs_v.at[j]], bp, sp)
        pltpu.async_copy(q_hbm.at[idxd_v.at[j]], bq, sq)

    def _wait_g(j, bp, bq, sp, sq):
        pltpu.make_async_copy(p_hbm.at[idxs_v.at[j]], bp, sp).wait()
        pltpu.make_async_copy(q_hbm.at[idxd_v.at[j]], bq, sq).wait()

    def _start_w(j, bp, bq, sp, sq):
        pltpu.async_copy(bp, rp_hbm.at[pl.ds(base + j * EBG, EBG)], sp)
        pltpu.async_copy(bq, rq_hbm.at[pl.ds(base + j * EBG, EBG)], sq)

    def _wait_w(j, bp, bq, sp, sq):
        pltpu.make_async_copy(bp, rp_hbm.at[pl.ds(base + j * EBG, EBG)],
                              sp).wait()
        pltpu.make_async_copy(bq, rq_hbm.at[pl.ds(base + j * EBG, EBG)],
                              sq).wait()

    _start_g(0, bufpa, bufqa, gpa, gqa)

    @pl.loop(0, NBG // 2)
    def _batches(j2):
        j = j2 * 2
        _wait_g(j, bufpa, bufqa, gpa, gqa)

        @pl.when(j2 > 0)
        def _():
            _wait_w(j - 1, bufpb, bufqb, wpb, wqb)

        _start_g(j + 1, bufpb, bufqb, gpb, gqb)
        _start_w(j, bufpa, bufqa, wpa, wqa)
        _wait_g(j + 1, bufpb, bufqb, gpb, gqb)
        _wait_w(j, bufpa, bufqa, wpa, wqa)

        @pl.when(j2 < NBG // 2 - 1)
        def _():
            _start_g(j + 2, bufpa, bufqa, gpa, gqa)

        _start_w(j + 1, bufpb, bufqb, wpb, wqb)

    _wait_w(NBG - 1, bufpb, bufqb, wpb, wqb)


_gather = pl.kernel(
    _gather_body,
    out_type=[jax.ShapeDtypeStruct((EP, H), jnp.float32),
              jax.ShapeDtypeStruct((EP, H), jnp.float32)],
    mesh=_mesh,
    scratch_types=[
        pltpu.VMEM((NBG, EBG), jnp.int32),
        pltpu.VMEM((NBG, EBG), jnp.int32),
        pltpu.VMEM((EBG, H), jnp.float32),
        pltpu.VMEM((EBG, H), jnp.float32),
        pltpu.VMEM((EBG, H), jnp.float32),
        pltpu.VMEM((EBG, H), jnp.float32),
    ] + [pltpu.SemaphoreType.DMA] * 8,
)


def _deg_inv_sqrt(hist_blk):
    deg = hist_blk[0, :, 0] + hist_blk[1, :, 0] + 1.0
    return lax.rsqrt(deg)


def _tc1_body(x_ref, hist_ref, w1_ref, o0, o1, o2, o3):
    dinv = _deg_inv_sqrt(hist_ref[...])
    h = jnp.dot(x_ref[...], w1_ref[...], preferred_element_type=jnp.float32)
    g = h * dinv[:, None]
    for c, o in enumerate((o0, o1, o2, o3)):
        o[...] = g[:, c * HC:(c + 1) * HC]


def _tc_mid_body(s_ref, g0, g1, g2, g3, hist_ref, w_ref, b_ref,
                 o0, o1, o2, o3):
    dinv = _deg_inv_sqrt(hist_ref[...])
    parts = [s_ref[c] + (g0, g1, g2, g3)[c][...] for c in range(NCH)]
    s = jnp.concatenate(parts, axis=1)
    h = jnp.maximum(s * dinv[:, None] + b_ref[...], 0.0)
    g = jnp.dot(h, w_ref[...], preferred_element_type=jnp.float32) * dinv[:, None]
    for c, o in enumerate((o0, o1, o2, o3)):
        o[...] = g[:, c * HC:(c + 1) * HC]


def _tc3_body(s_ref, g0, g1, g2, g3, hist_ref, m1a_ref, m1b_ref, b2_ref,
              bm1_ref, p_ref, q_ref):
    dinv = _deg_inv_sqrt(hist_ref[...])
    parts = [s_ref[c] + (g0, g1, g2, g3)[c][...] for c in range(NCH)]
    s = jnp.concatenate(parts, axis=1)
    h = jnp.maximum(s * dinv[:, None] + b2_ref[...], 0.0)
    p_ref[...] = jnp.dot(h, m1a_ref[...],
                         preferred_element_type=jnp.float32) + bm1_ref[...]
    q_ref[...] = jnp.dot(h, m1b_ref[...], preferred_element_type=jnp.float32)


def _tc4_body(rp_ref, rq_ref, m2_ref, bm2_ref, m3_ref, bm3_ref, o_ref):
    e = jnp.maximum(rp_ref[...] + rq_ref[...], 0.0)
    f = jnp.maximum(
        jnp.dot(e, m2_ref[...], preferred_element_type=jnp.float32)
        + bm2_ref[...], 0.0)
    o_ref[...] = jnp.sum(f * m3_ref[...], axis=1, keepdims=True) + bm3_ref[...]


def _row_spec(shape2):
    return pl.BlockSpec(shape2, lambda i: (i, 0))


def _full_spec(shape):
    nd = len(shape)
    return pl.BlockSpec(shape, lambda i: (0,) * nd)


_gchunk_specs = [_row_spec((BR, HC)) for _ in range(NCH)]
_gchunk_out = [jax.ShapeDtypeStruct((NP, HC), jnp.float32) for _ in range(NCH)]

_tc1 = pl.pallas_call(
    _tc1_body,
    grid=(GRID_N,),
    in_specs=[
        _row_spec((BR, DIN)),
        pl.BlockSpec((NC, BR, HC), lambda i: (0, i, 0)),
        _full_spec((DIN, H)),
    ],
    out_specs=_gchunk_specs,
    out_shape=_gchunk_out,
)

_tc_mid = pl.pallas_call(
    _tc_mid_body,
    grid=(GRID_N,),
    in_specs=[
        pl.BlockSpec((NCH, BR, HC), lambda i: (0, i, 0)),
        *_gchunk_specs,
        pl.BlockSpec((NC, BR, HC), lambda i: (0, i, 0)),
        _full_spec((H, H)),
        _full_spec((1, H)),
    ],
    out_specs=_gchunk_specs,
    out_shape=_gchunk_out,
)

_tc3 = pl.pallas_call(
    _tc3_body,
    grid=(GRID_N,),
    in_specs=[
        pl.BlockSpec((NCH, BR, HC), lambda i: (0, i, 0)),
        *_gchunk_specs,
        pl.BlockSpec((NC, BR, HC), lambda i: (0, i, 0)),
        _full_spec((H, H)),
        _full_spec((H, H)),
        _full_spec((1, H)),
        _full_spec((1, H)),
    ],
    out_specs=[_row_spec((BR, H)), _row_spec((BR, H))],
    out_shape=[jax.ShapeDtypeStruct((NP, H), jnp.float32),
               jax.ShapeDtypeStruct((NP, H), jnp.float32)],
)

_tc4 = pl.pallas_call(
    _tc4_body,
    grid=(GRID_E,),
    in_specs=[
        _row_spec((BE, H)),
        _row_spec((BE, H)),
        _full_spec((H, H // 2)),
        _full_spec((1, H // 2)),
        _full_spec((1, H // 2)),
        _full_spec((1, 1)),
    ],
    out_specs=_row_spec((BE, 1)),
    out_shape=jax.ShapeDtypeStruct((EP, 1), jnp.float32),
)


def kernel(x, edge_index, W1, b1, W2, b2, M1, bm1, M2, bm2, M3, bm3):
    src = edge_index[0].astype(jnp.int32)
    dst = edge_index[1].astype(jnp.int32)
    pad = jnp.full((EP - E,), N, jnp.int32)
    src3 = jnp.concatenate([src, pad]).reshape(NW, NB, EB)
    dst3 = jnp.concatenate([dst, pad]).reshape(NW, NB, EB)
    xp = jnp.pad(x, ((0, NP - N), (0, 0)))
    ones_h = jnp.ones((EB, HC), jnp.float32)
    zc = jnp.zeros((NP, HC), jnp.float32)
    b1r = b1.reshape(1, H)
    b2r = b2.reshape(1, H)
    bm1r = bm1.reshape(1, H)
    bm2r = bm2.reshape(1, H // 2)
    m3r = M3.reshape(1, H // 2)
    bm3r = bm3.reshape(1, 1)
    M1a = M1[:H]
    M1b = M1[H:]

    hist = _hist(dst3, ones_h, zc)

    g1 = _tc1(xp, hist, W1)
    s1 = _scatter(*g1, src3, dst3, zc)
    g2 = _tc_mid(s1, *g1, hist, W2, b1r)
    s2 = _scatter(*g2, src3, dst3, zc)
    p, q = _tc3(s2, *g2, hist, M1a, M1b, b2r, bm1r)
    srcg = src3.reshape(NW, NBG, EBG)
    dstg = dst3.reshape(NW, NBG, EBG)
    rp, rq = _gather(p, q, srcg, dstg)
    out = _tc4(rp, rq, M2, bm2r, m3r, bm3r)
    return out[:E]


# speedup vs baseline: 1.1916x; 1.1916x over previous
import functools

import jax
import jax.numpy as jnp
from jax import lax
from jax.experimental import pallas as pl
from jax.experimental.pallas import tpu as pltpu
from jax.experimental.pallas import tpu_sc as plsc

N = 10000
NP = 10112
DIN = 256
H = 512
HC = 128
NCH = H // HC
E = 160000
NC = 2
NS = 16
NW = NC * NS
EPT = 5120
EP = NW * EPT
EB = 128
NB = EPT // EB
RPT = NP // NS

EBG = 32
NB0 = 64
NB1 = 16
NBP = NB0 * NS + NB1 * (NS - 1) + NB0
NBG = EPT // EBG

BR = 1264
GRID_N = NP // BR
BE = 2048
GRID_E = EP // BE

_mesh = plsc.VectorSubcoreMesh(core_axis_name="c", subcore_axis_name="s",
                               num_cores=NC, num_subcores=NS)


def _hist_body(dst_hbm, ones_hbm, zv_hbm, hist_hbm, idx_v, ones_v, acc):
    cid = lax.axis_index("c")
    sid = lax.axis_index("s")
    wid = sid * NC + cid
    pltpu.sync_copy(dst_hbm.at[wid], idx_v)
    pltpu.sync_copy(ones_hbm, ones_v)
    pltpu.sync_copy(zv_hbm.at[pl.ds(sid * RPT, RPT)],
                    acc.at[pl.ds(sid * RPT, RPT)])
    plsc.subcore_barrier()

    @pl.loop(0, NB)
    def _batches(j):
        pltpu.sync_copy(ones_v, acc.at[idx_v.at[j]], add=True)

    plsc.subcore_barrier()
    pltpu.sync_copy(acc.at[pl.ds(sid * RPT, RPT)],
                    hist_hbm.at[cid, pl.ds(sid * RPT, RPT)])


_hist = pl.kernel(
    _hist_body,
    out_type=jax.ShapeDtypeStruct((NC, NP, HC), jnp.float32),
    mesh=_mesh,
    scratch_types=[
        pltpu.VMEM((NB, EB), jnp.int32),
        pltpu.VMEM((EB, HC), jnp.float32),
        pltpu.VMEM_SHARED((NP, HC), jnp.float32),
    ],
)


def _scatter_body(g0, g1, g2, g3, src_hbm, dst_hbm, z_hbm, s_hbm,
                  idxs_v, idxd_v, bufa, bufb, acc, sema, semb):
    cid = lax.axis_index("c")
    sid = lax.axis_index("s")
    start = jnp.where(cid == 0, sid * NB0, NB0 * NS + sid * NB1)
    nb2 = jnp.where(cid == 0, NB0 // 2, NB1 // 2)
    pltpu.sync_copy(src_hbm.at[pl.ds(start, NB0)], idxs_v)
    pltpu.sync_copy(dst_hbm.at[pl.ds(start, NB0)], idxd_v)
    for c, gc in enumerate((g0, g1, g2, g3)):
        pltpu.sync_copy(z_hbm.at[pl.ds(sid * RPT, RPT)],
                        acc.at[pl.ds(sid * RPT, RPT)])
        plsc.subcore_barrier()

        pltpu.async_copy(gc.at[idxs_v.at[0]], bufa, sema)

        @pl.loop(0, nb2)
        def _batches(j2):
            j = j2 * 2
            pltpu.make_async_copy(gc.at[idxs_v.at[j]], bufa, sema).wait()
            pltpu.async_copy(gc.at[idxs_v.at[j + 1]], bufb, semb)
            pltpu.sync_copy(bufa, acc.at[idxd_v.at[j]], add=True)
            pltpu.make_async_copy(gc.at[idxs_v.at[j + 1]], bufb, semb).wait()

            @pl.when(j2 < nb2 - 1)
            def _():
                pltpu.async_copy(gc.at[idxs_v.at[j + 2]], bufa, sema)

            pltpu.sync_copy(bufb, acc.at[idxd_v.at[j + 1]], add=True)

        plsc.subcore_barrier()
        pltpu.sync_copy(acc.at[pl.ds(sid * RPT, RPT)],
                        s_hbm.at[c, cid, pl.ds(sid * RPT, RPT)])
        plsc.subcore_barrier()


_scatter = pl.kernel(
    _scatter_body,
    out_type=jax.ShapeDtypeStruct((NCH, NC, NP, HC), jnp.float32),
    mesh=_mesh,
    scratch_types=[
        pltpu.VMEM((NB0, EB), jnp.int32),
        pltpu.VMEM((NB0, EB), jnp.int32),
        pltpu.VMEM((EB, HC), jnp.float32),
        pltpu.VMEM((EB, HC), jnp.float32),
        pltpu.VMEM_SHARED((NP, HC), jnp.float32),
        pltpu.SemaphoreType.DMA,
        pltpu.SemaphoreType.DMA,
    ],
)


def _gather_body(p_hbm, q_hbm, src_hbm, dst_hbm, rp_hbm, rq_hbm,
                 idxs_v, idxd_v, bufpa, bufqa, bufpb, bufqb,
                 gpa, gqa, gpb, gqb, wpa, wqa, wpb, wqb):
    cid = lax.axis_index("c")
    sid = lax.axis_index("s")
    wid = sid * NC + cid
    pltpu.sync_copy(src_hbm.at[wid], idxs_v)
    pltpu.sync_copy(dst_hbm.at[wid], idxd_v)
    base = wid * EPT

    def _start_g(j, bp, bq, sp, sq):
        pltpu.async_copy(p_hbm.at[idxs_v.at[j]], bp, sp)
        pltpu.async_copy(q_hbm.at[idxd_v.at[j]], bq, sq)

    def _wait_g(j, bp, bq, sp, sq):
        pltpu.make_async_copy(p_hbm.at[idxs_v.at[j]], bp, sp).wait()
        pltpu.make_async_copy(q_hbm.at[idxd_v.at[j]], bq, sq).wait()

    def _start_w(j, bp, bq, sp, sq):
        pltpu.async_copy(bp, rp_hbm.at[pl.ds(base + j * EBG, EBG)], sp)
        pltpu.async_copy(bq, rq_hbm.at[pl.ds(base + j * EBG, EBG)], sq)

    def _wait_w(j, bp, bq, sp, sq):
        pltpu.make_async_copy(bp, rp_hbm.at[pl.ds(base + j * EBG, EBG)],
                              sp).wait()
        pltpu.make_async_copy(bq, rq_hbm.at[pl.ds(base + j * EBG, EBG)],
                              sq).wait()

    _start_g(0, bufpa, bufqa, gpa, gqa)

    @pl.loop(0, NBG // 2)
    def _batches(j2):
        j = j2 * 2
        _wait_g(j, bufpa, bufqa, gpa, gqa)

        @pl.when(j2 > 0)
        def _():
            _wait_w(j - 1, bufpb, bufqb, wpb, wqb)

        _start_g(j + 1, bufpb, bufqb, gpb, gqb)
        _start_w(j, bufpa, bufqa, wpa, wqa)
        _wait_g(j + 1, bufpb, bufqb, gpb, gqb)
        _wait_w(j, bufpa, bufqa, wpa, wqa)

        @pl.when(j2 < NBG // 2 - 1)
        def _():
            _start_g(j + 2, bufpa, bufqa, gpa, gqa)

        _start_w(j + 1, bufpb, bufqb, wpb, wqb)

    _wait_w(NBG - 1, bufpb, bufqb, wpb, wqb)


_gather = pl.kernel(
    _gather_body,
    out_type=[jax.ShapeDtypeStruct((EP, H), jnp.float32),
              jax.ShapeDtypeStruct((EP, H), jnp.float32)],
    mesh=_mesh,
    scratch_types=[
        pltpu.VMEM((NBG, EBG), jnp.int32),
        pltpu.VMEM((NBG, EBG), jnp.int32),
        pltpu.VMEM((EBG, H), jnp.float32),
        pltpu.VMEM((EBG, H), jnp.float32),
        pltpu.VMEM((EBG, H), jnp.float32),
        pltpu.VMEM((EBG, H), jnp.float32),
    ] + [pltpu.SemaphoreType.DMA] * 8,
)


def _deg_inv_sqrt(hist_blk):
    deg = hist_blk[0, :, 0] + hist_blk[1, :, 0] + 1.0
    return lax.rsqrt(deg)


def _tc1_body(x_ref, hist_ref, w1_ref, o0, o1, o2, o3):
    dinv = _deg_inv_sqrt(hist_ref[...])
    h = jnp.dot(x_ref[...], w1_ref[...], preferred_element_type=jnp.float32)
    g = h * dinv[:, None]
    for c, o in enumerate((o0, o1, o2, o3)):
        o[...] = g[:, c * HC:(c + 1) * HC]


def _tc_mid_body(s_ref, g0, g1, g2, g3, hist_ref, w_ref, b_ref,
                 o0, o1, o2, o3):
    dinv = _deg_inv_sqrt(hist_ref[...])
    parts = [s_ref[c, 0] + s_ref[c, 1] + (g0, g1, g2, g3)[c][...]
             for c in range(NCH)]
    s = jnp.concatenate(parts, axis=1)
    h = jnp.maximum(s * dinv[:, None] + b_ref[...], 0.0)
    g = jnp.dot(h, w_ref[...], preferred_element_type=jnp.float32) * dinv[:, None]
    for c, o in enumerate((o0, o1, o2, o3)):
        o[...] = g[:, c * HC:(c + 1) * HC]


def _tc3_body(s_ref, g0, g1, g2, g3, hist_ref, m1a_ref, m1b_ref, b2_ref,
              bm1_ref, p_ref, q_ref):
    dinv = _deg_inv_sqrt(hist_ref[...])
    parts = [s_ref[c, 0] + s_ref[c, 1] + (g0, g1, g2, g3)[c][...]
             for c in range(NCH)]
    s = jnp.concatenate(parts, axis=1)
    h = jnp.maximum(s * dinv[:, None] + b2_ref[...], 0.0)
    p_ref[...] = jnp.dot(h, m1a_ref[...],
                         preferred_element_type=jnp.float32) + bm1_ref[...]
    q_ref[...] = jnp.dot(h, m1b_ref[...], preferred_element_type=jnp.float32)


def _tc4_body(rp_ref, rq_ref, m2_ref, bm2_ref, m3_ref, bm3_ref, o_ref):
    e = jnp.maximum(rp_ref[...] + rq_ref[...], 0.0)
    f = jnp.maximum(
        jnp.dot(e, m2_ref[...], preferred_element_type=jnp.float32)
        + bm2_ref[...], 0.0)
    o_ref[...] = jnp.sum(f * m3_ref[...], axis=1, keepdims=True) + bm3_ref[...]


def _row_spec(shape2):
    return pl.BlockSpec(shape2, lambda i: (i, 0))


def _full_spec(shape):
    nd = len(shape)
    return pl.BlockSpec(shape, lambda i: (0,) * nd)


_gchunk_specs = [_row_spec((BR, HC)) for _ in range(NCH)]
_gchunk_out = [jax.ShapeDtypeStruct((NP, HC), jnp.float32) for _ in range(NCH)]

_tc1 = pl.pallas_call(
    _tc1_body,
    grid=(GRID_N,),
    in_specs=[
        _row_spec((BR, DIN)),
        pl.BlockSpec((NC, BR, HC), lambda i: (0, i, 0)),
        _full_spec((DIN, H)),
    ],
    out_specs=_gchunk_specs,
    out_shape=_gchunk_out,
)

_tc_mid = pl.pallas_call(
    _tc_mid_body,
    grid=(GRID_N,),
    in_specs=[
        pl.BlockSpec((NCH, NC, BR, HC), lambda i: (0, 0, i, 0)),
        *_gchunk_specs,
        pl.BlockSpec((NC, BR, HC), lambda i: (0, i, 0)),
        _full_spec((H, H)),
        _full_spec((1, H)),
    ],
    out_specs=_gchunk_specs,
    out_shape=_gchunk_out,
)

_tc3 = pl.pallas_call(
    _tc3_body,
    grid=(GRID_N,),
    in_specs=[
        pl.BlockSpec((NCH, NC, BR, HC), lambda i: (0, 0, i, 0)),
        *_gchunk_specs,
        pl.BlockSpec((NC, BR, HC), lambda i: (0, i, 0)),
        _full_spec((H, H)),
        _full_spec((H, H)),
        _full_spec((1, H)),
        _full_spec((1, H)),
    ],
    out_specs=[_row_spec((BR, H)), _row_spec((BR, H))],
    out_shape=[jax.ShapeDtypeStruct((NP, H), jnp.float32),
               jax.ShapeDtypeStruct((NP, H), jnp.float32)],
)

_tc4 = pl.pallas_call(
    _tc4_body,
    grid=(GRID_E,),
    in_specs=[
        _row_spec((BE, H)),
        _row_spec((BE, H)),
        _full_spec((H, H // 2)),
        _full_spec((1, H // 2)),
        _full_spec((1, H // 2)),
        _full_spec((1, 1)),
    ],
    out_specs=_row_spec((BE, 1)),
    out_shape=jax.ShapeDtypeStruct((EP, 1), jnp.float32),
)


def kernel(x, edge_index, W1, b1, W2, b2, M1, bm1, M2, bm2, M3, bm3):
    src = edge_index[0].astype(jnp.int32)
    dst = edge_index[1].astype(jnp.int32)
    pad = jnp.full((EP - E,), N, jnp.int32)
    src3 = jnp.concatenate([src, pad]).reshape(NW, NB, EB)
    dst3 = jnp.concatenate([dst, pad]).reshape(NW, NB, EB)
    padf = jnp.full((NBP * EB - E,), N, jnp.int32)
    src4 = jnp.concatenate([src, padf]).reshape(NBP, EB)
    dst4 = jnp.concatenate([dst, padf]).reshape(NBP, EB)
    xp = jnp.pad(x, ((0, NP - N), (0, 0)))
    ones_h = jnp.ones((EB, HC), jnp.float32)
    zc = jnp.zeros((NP, HC), jnp.float32)
    b1r = b1.reshape(1, H)
    b2r = b2.reshape(1, H)
    bm1r = bm1.reshape(1, H)
    bm2r = bm2.reshape(1, H // 2)
    m3r = M3.reshape(1, H // 2)
    bm3r = bm3.reshape(1, 1)
    M1a = M1[:H]
    M1b = M1[H:]

    hist = _hist(dst3, ones_h, zc)

    g1 = _tc1(xp, hist, W1)
    s1 = _scatter(*g1, src4, dst4, zc)
    g2 = _tc_mid(s1, *g1, hist, W2, b1r)
    s2 = _scatter(*g2, src4, dst4, zc)
    p, q = _tc3(s2, *g2, hist, M1a, M1b, b2r, bm1r)
    srcg = src3.reshape(NW, NBG, EBG)
    dstg = dst3.reshape(NW, NBG, EBG)
    rp, rq = _gather(p, q, srcg, dstg)
    out = _tc4(rp, rq, M2, bm2r, m3r, bm3r)
    return out[:E]
